# baseline (speedup 1.0000x reference)
import jax
import jax.numpy as jnp
from jax.experimental import pallas as pl
from jax.experimental.pallas import tpu as pltpu

NUM_EMB = 4
EMB_DIM = 16
RANK = 8

BB = 256


def _body(idx_ref, u_ref, v_ref, out_ref):
    E = jnp.dot(u_ref[...], v_ref[...], preferred_element_type=jnp.float32)
    idx3 = idx_ref[...][:, :, None]
    e0 = E[0, :][None, None, :]
    e1 = E[1, :][None, None, :]
    e2 = E[2, :][None, None, :]
    e3 = E[3, :][None, None, :]
    out_ref[...] = jnp.where(
        idx3 < 2,
        jnp.where(idx3 == 0, e0, e1),
        jnp.where(idx3 == 2, e2, e3),
    )


def kernel(idx, U, V):
    B, Lseq = idx.shape
    idx32 = idx.astype(jnp.int32)
    grid = (B // BB,)
    return pl.pallas_call(
        _body,
        grid=grid,
        in_specs=[
            pl.BlockSpec((BB, Lseq), lambda i: (i, 0)),
            pl.BlockSpec((NUM_EMB, RANK), lambda i: (0, 0)),
            pl.BlockSpec((RANK, EMB_DIM), lambda i: (0, 0)),
        ],
        out_specs=pl.BlockSpec((BB, Lseq, EMB_DIM), lambda i: (i, 0, 0)),
        out_shape=jax.ShapeDtypeStruct((B, Lseq, EMB_DIM), jnp.float32),
        compiler_params=pltpu.CompilerParams(
            dimension_semantics=("arbitrary",),
            vmem_limit_bytes=115 * 1024 * 1024,
        ),
    )(idx32, U, V)
